# baseline (speedup 1.0000x reference)
import jax
import jax.numpy as jnp
from jax import lax
from jax.experimental import pallas as pl
from jax.experimental.pallas import tpu as pltpu
from jax.experimental.pallas import tpu_sc as plsc

B = 4096
K = 10
D = 64
NU = 100000
NI = 100000
F = 10
UFEAT = 3207
IFEAT = 2094

NC = 2
NS = 16
NW = NC * NS
NU_PAD = 100352
N_PER_W = NU_PAD // NW
CHUNK = 56
NCHUNK = N_PER_W // CHUNK
UFEAT_PAD = 3208
IFEAT_PAD = 2096

_mesh = plsc.VectorSubcoreMesh(core_axis_name="c", subcore_axis_name="s")
_sc_params = pltpu.CompilerParams(use_tc_tiling_on_sc=False, needs_layout_passes=False)


def _wid():
  return lax.axis_index("s") * NC + lax.axis_index("c")


def _fire(tbl, idx_v, rows_v, sem, nrows):
  for g in range(nrows // 80):
    pltpu.async_copy(tbl.at[idx_v.at[pl.ds(g * 80, 80)]],
                     rows_v.at[pl.ds(g * 80, 80)], sem)


def _drain(tbl, idx_v, rows_v, sem, nrows):
  for g in range(nrows // 80):
    pltpu.make_async_copy(tbl.at[idx_v.at[pl.ds(g * 80, 80)]],
                          rows_v.at[pl.ds(g * 80, 80)], sem).wait()


def _tree(vals):
  while len(vals) > 1:
    nxt = [vals[i] + vals[i + 1] for i in range(0, len(vals) - 1, 2)]
    if len(vals) % 2:
      nxt.append(vals[-1])
    vals = nxt
  return vals[0]


def _reduce(rows_v, out_v, nodes, r):
  def node(u, carry):
    base = u * r
    for h in range(D // 32):
      sl = pl.ds(h * 32, 32)
      acc_a = None
      acc_b = None
      for j0 in range(0, r, 8):
        terms = [plsc.unpack(rows_v[base + j, sl],
                             format=plsc.PackFormat.INTERLEAVED)
                 for j in range(j0, min(j0 + 8, r))]
        ta = _tree([t[0] for t in terms])
        tb = _tree([t[1] for t in terms])
        acc_a = ta if acc_a is None else acc_a + ta
        acc_b = tb if acc_b is None else acc_b + tb
      out_v[u, sl] = plsc.pack(acc_a, acc_b,
                               format=plsc.PackFormat.INTERLEAVED)
    return carry

  lax.fori_loop(0, nodes, node, 0)


def _gather_sum_pipeline(tbl, fidx, outp, bufs, *, nchunks, nodes, r,
                         idx0_fn, orow_fn):
  idx_b, rows_b, out_b, sem_b = bufs
  nrows = nodes * r

  def fetch(c, p):
    pltpu.sync_copy(fidx.at[pl.ds(idx0_fn(c), nrows)],
                    idx_b[p].at[pl.ds(0, nrows)])
    _fire(tbl, idx_b[p], rows_b[p], sem_b[p], nrows)

  def consume(c, p):
    _drain(tbl, idx_b[p], rows_b[p], sem_b[p], nrows)
    _reduce(rows_b[p], out_b[p], nodes, r)
    pltpu.sync_copy(out_b[p].at[pl.ds(0, nodes)],
                    outp.at[pl.ds(orow_fn(c), nodes)])

  fetch(0, 0)

  def pair(i, carry):
    c0 = 2 * i
    fetch(c0 + 1, 1)
    consume(c0, 0)

    @pl.when(c0 + 2 < nchunks)
    def _():
      fetch(c0 + 2, 0)

    consume(c0 + 1, 1)
    return carry

  lax.fori_loop(0, nchunks // 2, pair, 0)


BCHUNK = 112
BNCHUNK = N_PER_W // BCHUNK


def _bag_body(pu, pi, uidx, iidx, bagu, bagi,
              tbl_v, idx_a, idx_bb, out_a, out_bb, sem_a, sem_bb):
  wid = _wid()
  idx_b = (idx_a, idx_bb)
  out_b = (out_a, out_bb)
  sems = (sem_a, sem_bb)
  lane = lax.broadcasted_iota(jnp.int32, (16,), 0)
  lane_f = lane * F
  lane_w = lane * (D // 2)
  fmt = plsc.PackFormat.INTERLEAVED

  def run(tbl_hbm, twords, fidx, outp):
    pltpu.sync_copy(tbl_hbm, tbl_v.at[pl.ds(0, twords)])

    def fetch(c, p):
      i0 = (wid * N_PER_W + c * BCHUNK) * F
      pltpu.async_copy(fidx.at[pl.ds(i0, BCHUNK * F)], idx_b[p], sems[p])

    def consume(c, p):
      pltpu.make_async_copy(fidx.at[pl.ds(0, BCHUNK * F)], idx_b[p],
                            sems[p]).wait()

      def group(g, carry):
        u0 = g * 16
        base_pos = u0 * F + lane_f
        bases = [plsc.load_gather(idx_b[p], [base_pos + f]) * (D // 2)
                 for f in range(F)]
        obase = u0 * (D // 2) + lane_w
        for hw in range(D // 2):
          unp = [plsc.unpack(plsc.bitcast(
                     plsc.load_gather(tbl_v, [bases[f] + hw]), jnp.bfloat16),
                     format=fmt) for f in range(F)]
          aa = _tree([x[0] for x in unp])
          bb = _tree([x[1] for x in unp])
          packed = plsc.bitcast(plsc.pack(aa, bb, format=fmt), jnp.int32)
          plsc.store_scatter(out_b[p], [obase + hw], packed)
        return carry

      lax.fori_loop(0, BCHUNK // 16, group, 0)
      o0 = (wid * N_PER_W + c * BCHUNK) * (D // 2)
      pltpu.sync_copy(out_b[p], outp.at[pl.ds(o0, BCHUNK * (D // 2))])

    fetch(0, 0)

    def pair(i, carry):
      c0 = 2 * i
      fetch(c0 + 1, 1)
      consume(c0, 0)

      @pl.when(c0 + 2 < BNCHUNK)
      def _():
        fetch(c0 + 2, 0)

      consume(c0 + 1, 1)
      return carry

    lax.fori_loop(0, BNCHUNK // 2, pair, 0)

  run(pu, UFEAT_PAD * (D // 2), uidx, bagu)
  run(pi, IFEAT_PAD * (D // 2), iidx, bagi)


_bag_call = pl.kernel(
    _bag_body,
    out_type=(jax.ShapeDtypeStruct((NU_PAD * (D // 2),), jnp.int32),
              jax.ShapeDtypeStruct((NU_PAD * (D // 2),), jnp.int32)),
    mesh=_mesh,
    compiler_params=_sc_params,
    scratch_types=[
        pltpu.VMEM((UFEAT_PAD * (D // 2),), jnp.int32),
        pltpu.VMEM((BCHUNK * F,), jnp.int32),
        pltpu.VMEM((BCHUNK * F,), jnp.int32),
        pltpu.VMEM((BCHUNK * (D // 2),), jnp.int32),
        pltpu.VMEM((BCHUNK * (D // 2),), jnp.int32),
        pltpu.SemaphoreType.DMA,
        pltpu.SemaphoreType.DMA,
    ],
)


def _agg_body(bagu, bagi, n0r, n1r, n2r, h0s, g1s, t2s,
              idx_a, idx_bb, rows_a, rows_bb, out_a, out_bb, sem_a, sem_bb):
  wid = _wid()
  bufs = ((idx_a, idx_bb), (rows_a, rows_bb), (out_a, out_bb), (sem_a, sem_bb))

  _gather_sum_pipeline(
      bagu, n2r, t2s, bufs, nchunks=32, nodes=4, r=100,
      idx0_fn=lambda c: (wid * 128 + c * 4) * 100,
      orow_fn=lambda c: wid * 128 + c * 4)

  _gather_sum_pipeline(
      bagi, n1r, g1s, bufs, nchunks=16, nodes=8, r=K,
      idx0_fn=lambda c: (wid * 128 + c * 8) * K,
      orow_fn=lambda c: wid * 128 + c * 8)

  pltpu.sync_copy(n0r.at[pl.ds(wid * 128, 128)], idx_a.at[pl.ds(0, 128)])
  for g in range(2):
    pltpu.async_copy(bagu.at[idx_a.at[pl.ds(g * 64, 64)]],
                     rows_a.at[pl.ds(g * 64, 64)], sem_a)
  for g in range(2):
    pltpu.make_async_copy(bagu.at[idx_a.at[pl.ds(g * 64, 64)]],
                          rows_a.at[pl.ds(g * 64, 64)], sem_a).wait()
  pltpu.sync_copy(rows_a.at[pl.ds(0, 128)], h0s.at[pl.ds(wid * 128, 128)])


_agg_call = pl.kernel(
    _agg_body,
    out_type=(jax.ShapeDtypeStruct((B, D), jnp.bfloat16),
              jax.ShapeDtypeStruct((B, D), jnp.bfloat16),
              jax.ShapeDtypeStruct((B, D), jnp.bfloat16)),
    mesh=_mesh,
    compiler_params=_sc_params,
    scratch_types=[
        pltpu.VMEM((400,), jnp.int32),
        pltpu.VMEM((400,), jnp.int32),
        pltpu.VMEM((400, D), jnp.bfloat16),
        pltpu.VMEM((400, D), jnp.bfloat16),
        pltpu.VMEM((8, D), jnp.bfloat16),
        pltpu.VMEM((8, D), jnp.bfloat16),
        pltpu.SemaphoreType.DMA,
        pltpu.SemaphoreType.DMA,
    ],
)


def _dg(a, b):
  return lax.dot_general(a, b, (((1,), (1,)), ((), ())),
                         preferred_element_type=jnp.float32)


def _proj_body(e_ref, w_ref, o_ref):
  o_ref[...] = (_dg(e_ref[...], w_ref[...]) * (1.0 / F)).astype(jnp.bfloat16)


def _proj(e, w):
  rows = e.shape[0]
  pad = (-rows) % 8
  e = jnp.pad(e, ((0, pad), (0, 0)))
  return pl.pallas_call(
      _proj_body,
      out_shape=jax.ShapeDtypeStruct((rows + pad, D), jnp.bfloat16),
  )(e, w)


def _final_body(h0_ref, g1_ref, t2_ref, w0_ref, w1_ref,
                bu_ref, bi_ref, b0_ref, b1_ref, o_ref):
  h0 = h0_ref[...].astype(jnp.float32) + bu_ref[...]
  g1 = g1_ref[...].astype(jnp.float32) + float(K) * bi_ref[...]
  t2 = t2_ref[...].astype(jnp.float32) + float(K * K) * bu_ref[...]
  w0 = w0_ref[...]
  w1 = w1_ref[...]
  w0a, w0b = w0[:, :D], w0[:, D:]
  w1a, w1b = w1[:, :D], w1[:, D:]
  y0 = _dg(h0, w0a) + _dg(g1, w0b) + b0_ref[...]
  z = _dg(g1, w0a) + _dg(t2, w0b) + float(K) * b0_ref[...]
  o_ref[...] = _dg(y0, w1a) + _dg(z, w1b) + b1_ref[...]


_final = pl.pallas_call(
    _final_body,
    out_shape=jax.ShapeDtypeStruct((B, D), jnp.float32),
)


@jax.jit
def kernel(n0, n1, n2, user_feat_idx, item_feat_idx, user_feat_emb,
           item_feat_emb, user_proj_w, user_proj_b, item_proj_w, item_proj_b,
           w0_w, w0_b, w1_w, w1_b):
  pu = _proj(user_feat_emb, user_proj_w)
  pi = _proj(item_feat_emb, item_proj_w)
  pu_i = lax.bitcast_convert_type(pu.reshape(-1, 2), jnp.int32).reshape(-1)
  pi_i = lax.bitcast_convert_type(pi.reshape(-1, 2), jnp.int32).reshape(-1)
  uidx = jnp.pad(user_feat_idx, (0, (NU_PAD - NU) * F))
  iidx = jnp.pad(item_feat_idx, (0, (NU_PAD - NI) * F))
  bagu_i, bagi_i = _bag_call(pu_i, pi_i, uidx, iidx)
  bagu = lax.bitcast_convert_type(
      bagu_i.reshape(NU_PAD, D // 2), jnp.bfloat16).reshape(NU_PAD, D)
  bagi = lax.bitcast_convert_type(
      bagi_i.reshape(NU_PAD, D // 2), jnp.bfloat16).reshape(NU_PAD, D)
  h0s, g1s, t2s = _agg_call(bagu, bagi, n0, n1, n2)
  return _final(h0s, g1s, t2s, w0_w, w1_w,
                user_proj_b.reshape(1, D), item_proj_b.reshape(1, D),
                w0_b.reshape(1, D), w1_b.reshape(1, D))


# speedup vs baseline: 5.3531x; 5.3531x over previous
import jax
import jax.numpy as jnp
from jax import lax
from jax.experimental import pallas as pl
from jax.experimental.pallas import tpu as pltpu
from jax.experimental.pallas import tpu_sc as plsc

B = 4096
K = 10
D = 64
NU = 100000
NI = 100000
F = 10
UFEAT = 3207
IFEAT = 2094

NC = 2
NS = 16
NW = NC * NS
NU_PAD = 100352
N_PER_W = NU_PAD // NW
CHUNK = 56
NCHUNK = N_PER_W // CHUNK
UFEAT_PAD = 3208
IFEAT_PAD = 2096

_mesh = plsc.VectorSubcoreMesh(core_axis_name="c", subcore_axis_name="s")
_sc_params = pltpu.CompilerParams(use_tc_tiling_on_sc=False, needs_layout_passes=False)


def _wid():
  return lax.axis_index("s") * NC + lax.axis_index("c")


def _fire(tbl, idx_v, rows_v, sem, nrows):
  for g in range(nrows // 80):
    pltpu.async_copy(tbl.at[idx_v.at[pl.ds(g * 80, 80)]],
                     rows_v.at[pl.ds(g * 80, 80)], sem)


def _drain(tbl, idx_v, rows_v, sem, nrows):
  for g in range(nrows // 80):
    pltpu.make_async_copy(tbl.at[idx_v.at[pl.ds(g * 80, 80)]],
                          rows_v.at[pl.ds(g * 80, 80)], sem).wait()


def _tree(vals):
  while len(vals) > 1:
    nxt = [vals[i] + vals[i + 1] for i in range(0, len(vals) - 1, 2)]
    if len(vals) % 2:
      nxt.append(vals[-1])
    vals = nxt
  return vals[0]


def _reduce(rows_v, out_v, nodes, r):
  def node(u, carry):
    base = u * r
    for h in range(D // 32):
      sl = pl.ds(h * 32, 32)
      acc_a = None
      acc_b = None
      for j0 in range(0, r, 8):
        terms = [plsc.unpack(rows_v[base + j, sl],
                             format=plsc.PackFormat.INTERLEAVED)
                 for j in range(j0, min(j0 + 8, r))]
        ta = _tree([t[0] for t in terms])
        tb = _tree([t[1] for t in terms])
        acc_a = ta if acc_a is None else acc_a + ta
        acc_b = tb if acc_b is None else acc_b + tb
      out_v[u, sl] = plsc.pack(acc_a, acc_b,
                               format=plsc.PackFormat.INTERLEAVED)
    return carry

  lax.fori_loop(0, nodes, node, 0)


def _gather_sum_pipeline(tbl, fidx, outp, bufs, *, nchunks, nodes, r,
                         idx0_fn, orow_fn):
  idx_b, rows_b, out_b, sem_b = bufs
  nrows = nodes * r

  def fetch(c, p):
    pltpu.sync_copy(fidx.at[pl.ds(idx0_fn(c), nrows)],
                    idx_b[p].at[pl.ds(0, nrows)])
    _fire(tbl, idx_b[p], rows_b[p], sem_b[p], nrows)

  def consume(c, p):
    _drain(tbl, idx_b[p], rows_b[p], sem_b[p], nrows)
    _reduce(rows_b[p], out_b[p], nodes, r)
    pltpu.sync_copy(out_b[p].at[pl.ds(0, nodes)],
                    outp.at[pl.ds(orow_fn(c), nodes)])

  fetch(0, 0)

  def pair(i, carry):
    c0 = 2 * i
    fetch(c0 + 1, 1)
    consume(c0, 0)

    @pl.when(c0 + 2 < nchunks)
    def _():
      fetch(c0 + 2, 0)

    consume(c0 + 1, 1)
    return carry

  lax.fori_loop(0, nchunks // 2, pair, 0)


def _bag_body(pu, pi, uidx, iidx, bagu, bagi, tblu_s, tbli_s,
              idx_a, idx_bb, rows_a, rows_bb, out_a, out_bb, sem_a, sem_bb):
  wid = _wid()
  bufs = ((idx_a, idx_bb), (rows_a, rows_bb), (out_a, out_bb), (sem_a, sem_bb))

  @pl.when(lax.axis_index("s") == 0)
  def _():
    pltpu.sync_copy(pu, tblu_s)
    pltpu.sync_copy(pi, tbli_s)

  plsc.subcore_barrier()

  def run(tbl, fidx, outp):
    _gather_sum_pipeline(
        tbl, fidx, outp, bufs, nchunks=NCHUNK, nodes=CHUNK, r=F,
        idx0_fn=lambda c: (wid * N_PER_W + c * CHUNK) * F,
        orow_fn=lambda c: wid * N_PER_W + c * CHUNK)

  run(tblu_s, uidx, bagu)
  run(tbli_s, iidx, bagi)


_bag_call = pl.kernel(
    _bag_body,
    out_type=(jax.ShapeDtypeStruct((NU_PAD, D), jnp.bfloat16),
              jax.ShapeDtypeStruct((NU_PAD, D), jnp.bfloat16)),
    mesh=_mesh,
    compiler_params=_sc_params,
    scratch_types=[
        pltpu.VMEM_SHARED((UFEAT_PAD, D), jnp.bfloat16),
        pltpu.VMEM_SHARED((IFEAT_PAD, D), jnp.bfloat16),
        pltpu.VMEM((CHUNK * F,), jnp.int32),
        pltpu.VMEM((CHUNK * F,), jnp.int32),
        pltpu.VMEM((CHUNK * F, D), jnp.bfloat16),
        pltpu.VMEM((CHUNK * F, D), jnp.bfloat16),
        pltpu.VMEM((CHUNK, D), jnp.bfloat16),
        pltpu.VMEM((CHUNK, D), jnp.bfloat16),
        pltpu.SemaphoreType.DMA,
        pltpu.SemaphoreType.DMA,
    ],
)


def _agg_body(bagu, bagi, n0r, n1r, n2r, h0s, g1s, t2s,
              idx_a, idx_bb, rows_a, rows_bb, out_a, out_bb, sem_a, sem_bb):
  wid = _wid()
  bufs = ((idx_a, idx_bb), (rows_a, rows_bb), (out_a, out_bb), (sem_a, sem_bb))

  _gather_sum_pipeline(
      bagu, n2r, t2s, bufs, nchunks=32, nodes=4, r=100,
      idx0_fn=lambda c: (wid * 128 + c * 4) * 100,
      orow_fn=lambda c: wid * 128 + c * 4)

  _gather_sum_pipeline(
      bagi, n1r, g1s, bufs, nchunks=16, nodes=8, r=K,
      idx0_fn=lambda c: (wid * 128 + c * 8) * K,
      orow_fn=lambda c: wid * 128 + c * 8)

  pltpu.sync_copy(n0r.at[pl.ds(wid * 128, 128)], idx_a.at[pl.ds(0, 128)])
  for g in range(2):
    pltpu.async_copy(bagu.at[idx_a.at[pl.ds(g * 64, 64)]],
                     rows_a.at[pl.ds(g * 64, 64)], sem_a)
  for g in range(2):
    pltpu.make_async_copy(bagu.at[idx_a.at[pl.ds(g * 64, 64)]],
                          rows_a.at[pl.ds(g * 64, 64)], sem_a).wait()
  pltpu.sync_copy(rows_a.at[pl.ds(0, 128)], h0s.at[pl.ds(wid * 128, 128)])


_agg_call = pl.kernel(
    _agg_body,
    out_type=(jax.ShapeDtypeStruct((B, D), jnp.bfloat16),
              jax.ShapeDtypeStruct((B, D), jnp.bfloat16),
              jax.ShapeDtypeStruct((B, D), jnp.bfloat16)),
    mesh=_mesh,
    compiler_params=_sc_params,
    scratch_types=[
        pltpu.VMEM((400,), jnp.int32),
        pltpu.VMEM((400,), jnp.int32),
        pltpu.VMEM((400, D), jnp.bfloat16),
        pltpu.VMEM((400, D), jnp.bfloat16),
        pltpu.VMEM((8, D), jnp.bfloat16),
        pltpu.VMEM((8, D), jnp.bfloat16),
        pltpu.SemaphoreType.DMA,
        pltpu.SemaphoreType.DMA,
    ],
)


def _dg(a, b):
  return lax.dot_general(a, b, (((1,), (1,)), ((), ())),
                         preferred_element_type=jnp.float32)


def _proj_body(e_ref, w_ref, o_ref):
  o_ref[...] = (_dg(e_ref[...], w_ref[...]) * (1.0 / F)).astype(jnp.bfloat16)


def _proj(e, w):
  rows = e.shape[0]
  pad = (-rows) % 8
  e = jnp.pad(e, ((0, pad), (0, 0)))
  return pl.pallas_call(
      _proj_body,
      out_shape=jax.ShapeDtypeStruct((rows + pad, D), jnp.bfloat16),
  )(e, w)


def _final_body(h0_ref, g1_ref, t2_ref, w0_ref, w1_ref,
                bu_ref, bi_ref, b0_ref, b1_ref, o_ref):
  h0 = h0_ref[...].astype(jnp.float32) + bu_ref[...]
  g1 = g1_ref[...].astype(jnp.float32) + float(K) * bi_ref[...]
  t2 = t2_ref[...].astype(jnp.float32) + float(K * K) * bu_ref[...]
  w0 = w0_ref[...]
  w1 = w1_ref[...]
  w0a, w0b = w0[:, :D], w0[:, D:]
  w1a, w1b = w1[:, :D], w1[:, D:]
  y0 = _dg(h0, w0a) + _dg(g1, w0b) + b0_ref[...]
  z = _dg(g1, w0a) + _dg(t2, w0b) + float(K) * b0_ref[...]
  o_ref[...] = _dg(y0, w1a) + _dg(z, w1b) + b1_ref[...]


_final = pl.pallas_call(
    _final_body,
    out_shape=jax.ShapeDtypeStruct((B, D), jnp.float32),
)


@jax.jit
def kernel(n0, n1, n2, user_feat_idx, item_feat_idx, user_feat_emb,
           item_feat_emb, user_proj_w, user_proj_b, item_proj_w, item_proj_b,
           w0_w, w0_b, w1_w, w1_b):
  pu = _proj(user_feat_emb, user_proj_w)
  pi = _proj(item_feat_emb, item_proj_w)
  uidx = jnp.pad(user_feat_idx, (0, (NU_PAD - NU) * F))
  iidx = jnp.pad(item_feat_idx, (0, (NU_PAD - NI) * F))
  bagu, bagi = _bag_call(pu, pi, uidx, iidx)
  h0s, g1s, t2s = _agg_call(bagu, bagi, n0, n1, n2)
  return _final(h0s, g1s, t2s, w0_w, w1_w,
                user_proj_b.reshape(1, D), item_proj_b.reshape(1, D),
                w0_b.reshape(1, D), w1_b.reshape(1, D))


# speedup vs baseline: 5.9593x; 1.1132x over previous
import jax
import jax.numpy as jnp
from jax import lax
from jax.experimental import pallas as pl
from jax.experimental.pallas import tpu as pltpu
from jax.experimental.pallas import tpu_sc as plsc

B = 4096
K = 10
D = 64
NU = 100000
NI = 100000
F = 10
UFEAT = 3207
IFEAT = 2094

NC = 2
NS = 16
NW = NC * NS
NU_PAD = 100352
N_PER_W = NU_PAD // NW
CHUNK = 112
NCHUNK = N_PER_W // CHUNK
UFEAT_PAD = 3208
IFEAT_PAD = 2096

_mesh = plsc.VectorSubcoreMesh(core_axis_name="c", subcore_axis_name="s")
_sc_params = pltpu.CompilerParams(use_tc_tiling_on_sc=False, needs_layout_passes=False)


def _wid():
  return lax.axis_index("s") * NC + lax.axis_index("c")


def _fire(tbl, idx_v, rows_v, sem, nrows):
  for g in range(nrows // 80):
    pltpu.async_copy(tbl.at[idx_v.at[pl.ds(g * 80, 80)]],
                     rows_v.at[pl.ds(g * 80, 80)], sem)


def _drain(tbl, idx_v, rows_v, sem, nrows):
  for g in range(nrows // 80):
    pltpu.make_async_copy(tbl.at[idx_v.at[pl.ds(g * 80, 80)]],
                          rows_v.at[pl.ds(g * 80, 80)], sem).wait()


def _tree(vals):
  while len(vals) > 1:
    nxt = [vals[i] + vals[i + 1] for i in range(0, len(vals) - 1, 2)]
    if len(vals) % 2:
      nxt.append(vals[-1])
    vals = nxt
  return vals[0]


def _reduce(rows_v, out_v, nodes, r):
  def node(u, carry):
    base = u * r
    for h in range(D // 32):
      sl = pl.ds(h * 32, 32)
      acc_a = None
      acc_b = None
      for j0 in range(0, r, 8):
        terms = [plsc.unpack(rows_v[base + j, sl],
                             format=plsc.PackFormat.INTERLEAVED)
                 for j in range(j0, min(j0 + 8, r))]
        ta = _tree([t[0] for t in terms])
        tb = _tree([t[1] for t in terms])
        acc_a = ta if acc_a is None else acc_a + ta
        acc_b = tb if acc_b is None else acc_b + tb
      out_v[u, sl] = plsc.pack(acc_a, acc_b,
                               format=plsc.PackFormat.INTERLEAVED)
    return carry

  lax.fori_loop(0, nodes, node, 0)


def _gather_sum_pipeline(tbl, fidx, outp, bufs, *, nchunks, nodes, r,
                         idx0_fn, orow_fn):
  idx_b, rows_b, out_b, sem_b = bufs
  nrows = nodes * r

  def fetch(c, p):
    pltpu.sync_copy(fidx.at[pl.ds(idx0_fn(c), nrows)],
                    idx_b[p].at[pl.ds(0, nrows)])
    _fire(tbl, idx_b[p], rows_b[p], sem_b[p], nrows)

  def consume(c, p):
    _drain(tbl, idx_b[p], rows_b[p], sem_b[p], nrows)
    _reduce(rows_b[p], out_b[p], nodes, r)
    pltpu.sync_copy(out_b[p].at[pl.ds(0, nodes)],
                    outp.at[pl.ds(orow_fn(c), nodes)])

  fetch(0, 0)

  def pair(i, carry):
    c0 = 2 * i
    fetch(c0 + 1, 1)
    consume(c0, 0)

    @pl.when(c0 + 2 < nchunks)
    def _():
      fetch(c0 + 2, 0)

    consume(c0 + 1, 1)
    return carry

  lax.fori_loop(0, nchunks // 2, pair, 0)


def _bag_body(pu, pi, uidx, iidx, bagu, bagi, tblu_s, tbli_s,
              idx_a, idx_bb, rows_a, rows_bb, out_a, out_bb, sem_a, sem_bb):
  wid = _wid()
  bufs = ((idx_a, idx_bb), (rows_a, rows_bb), (out_a, out_bb), (sem_a, sem_bb))

  @pl.when(lax.axis_index("s") == 0)
  def _():
    pltpu.sync_copy(pu, tblu_s)
    pltpu.sync_copy(pi, tbli_s)

  plsc.subcore_barrier()

  def run(tbl, fidx, outp):
    _gather_sum_pipeline(
        tbl, fidx, outp, bufs, nchunks=NCHUNK, nodes=CHUNK, r=F,
        idx0_fn=lambda c: (wid * N_PER_W + c * CHUNK) * F,
        orow_fn=lambda c: wid * N_PER_W + c * CHUNK)

  run(tblu_s, uidx, bagu)
  run(tbli_s, iidx, bagi)


_bag_call = pl.kernel(
    _bag_body,
    out_type=(jax.ShapeDtypeStruct((NU_PAD, D), jnp.bfloat16),
              jax.ShapeDtypeStruct((NU_PAD, D), jnp.bfloat16)),
    mesh=_mesh,
    compiler_params=_sc_params,
    scratch_types=[
        pltpu.VMEM_SHARED((UFEAT_PAD, D), jnp.bfloat16),
        pltpu.VMEM_SHARED((IFEAT_PAD, D), jnp.bfloat16),
        pltpu.VMEM((CHUNK * F,), jnp.int32),
        pltpu.VMEM((CHUNK * F,), jnp.int32),
        pltpu.VMEM((CHUNK * F, D), jnp.bfloat16),
        pltpu.VMEM((CHUNK * F, D), jnp.bfloat16),
        pltpu.VMEM((CHUNK, D), jnp.bfloat16),
        pltpu.VMEM((CHUNK, D), jnp.bfloat16),
        pltpu.SemaphoreType.DMA,
        pltpu.SemaphoreType.DMA,
    ],
)


def _agg_body(bagu, bagi, n0r, n1r, n2r, h0s, g1s, t2s,
              idx_a, idx_bb, rows_a, rows_bb, out_a, out_bb, sem_a, sem_bb):
  wid = _wid()
  bufs = ((idx_a, idx_bb), (rows_a, rows_bb), (out_a, out_bb), (sem_a, sem_bb))

  _gather_sum_pipeline(
      bagu, n2r, t2s, bufs, nchunks=16, nodes=8, r=100,
      idx0_fn=lambda c: (wid * 128 + c * 8) * 100,
      orow_fn=lambda c: wid * 128 + c * 8)

  _gather_sum_pipeline(
      bagi, n1r, g1s, bufs, nchunks=8, nodes=16, r=K,
      idx0_fn=lambda c: (wid * 128 + c * 16) * K,
      orow_fn=lambda c: wid * 128 + c * 16)

  pltpu.sync_copy(n0r.at[pl.ds(wid * 128, 128)], idx_a.at[pl.ds(0, 128)])
  for g in range(2):
    pltpu.async_copy(bagu.at[idx_a.at[pl.ds(g * 64, 64)]],
                     rows_a.at[pl.ds(g * 64, 64)], sem_a)
  for g in range(2):
    pltpu.make_async_copy(bagu.at[idx_a.at[pl.ds(g * 64, 64)]],
                          rows_a.at[pl.ds(g * 64, 64)], sem_a).wait()
  pltpu.sync_copy(rows_a.at[pl.ds(0, 128)], h0s.at[pl.ds(wid * 128, 128)])


_agg_call = pl.kernel(
    _agg_body,
    out_type=(jax.ShapeDtypeStruct((B, D), jnp.bfloat16),
              jax.ShapeDtypeStruct((B, D), jnp.bfloat16),
              jax.ShapeDtypeStruct((B, D), jnp.bfloat16)),
    mesh=_mesh,
    compiler_params=_sc_params,
    scratch_types=[
        pltpu.VMEM((800,), jnp.int32),
        pltpu.VMEM((800,), jnp.int32),
        pltpu.VMEM((800, D), jnp.bfloat16),
        pltpu.VMEM((800, D), jnp.bfloat16),
        pltpu.VMEM((16, D), jnp.bfloat16),
        pltpu.VMEM((16, D), jnp.bfloat16),
        pltpu.SemaphoreType.DMA,
        pltpu.SemaphoreType.DMA,
    ],
)


def _dg(a, b):
  return lax.dot_general(a, b, (((1,), (1,)), ((), ())),
                         preferred_element_type=jnp.float32)


def _proj_body(eu_ref, wu_ref, ei_ref, wi_ref, ou_ref, oi_ref):
  ou_ref[...] = (_dg(eu_ref[...], wu_ref[...]) * (1.0 / F)).astype(jnp.bfloat16)
  oi_ref[...] = (_dg(ei_ref[...], wi_ref[...]) * (1.0 / F)).astype(jnp.bfloat16)


def _proj2(eu, wu, ei, wi):
  eu = jnp.pad(eu, ((0, UFEAT_PAD - UFEAT), (0, 0)))
  ei = jnp.pad(ei, ((0, IFEAT_PAD - IFEAT), (0, 0)))
  return pl.pallas_call(
      _proj_body,
      out_shape=(jax.ShapeDtypeStruct((UFEAT_PAD, D), jnp.bfloat16),
                 jax.ShapeDtypeStruct((IFEAT_PAD, D), jnp.bfloat16)),
  )(eu, wu, ei, wi)


def _final_body(h0_ref, g1_ref, t2_ref, w0_ref, w1_ref,
                bu_ref, bi_ref, b0_ref, b1_ref, o_ref):
  h0 = h0_ref[...].astype(jnp.float32) + bu_ref[...]
  g1 = g1_ref[...].astype(jnp.float32) + float(K) * bi_ref[...]
  t2 = t2_ref[...].astype(jnp.float32) + float(K * K) * bu_ref[...]
  w0 = w0_ref[...]
  w1 = w1_ref[...]
  w0a, w0b = w0[:, :D], w0[:, D:]
  w1a, w1b = w1[:, :D], w1[:, D:]
  y0 = _dg(h0, w0a) + _dg(g1, w0b) + b0_ref[...]
  z = _dg(g1, w0a) + _dg(t2, w0b) + float(K) * b0_ref[...]
  o_ref[...] = _dg(y0, w1a) + _dg(z, w1b) + b1_ref[...]


_final = pl.pallas_call(
    _final_body,
    out_shape=jax.ShapeDtypeStruct((B, D), jnp.float32),
)


@jax.jit
def kernel(n0, n1, n2, user_feat_idx, item_feat_idx, user_feat_emb,
           item_feat_emb, user_proj_w, user_proj_b, item_proj_w, item_proj_b,
           w0_w, w0_b, w1_w, w1_b):
  pu, pi = _proj2(user_feat_emb, user_proj_w, item_feat_emb, item_proj_w)
  uidx = jnp.pad(user_feat_idx, (0, (NU_PAD - NU) * F))
  iidx = jnp.pad(item_feat_idx, (0, (NU_PAD - NI) * F))
  bagu, bagi = _bag_call(pu, pi, uidx, iidx)
  h0s, g1s, t2s = _agg_call(bagu, bagi, n0, n1, n2)
  return _final(h0s, g1s, t2s, w0_w, w1_w,
                user_proj_b.reshape(1, D), item_proj_b.reshape(1, D),
                w0_b.reshape(1, D), w1_b.reshape(1, D))


# speedup vs baseline: 6.5438x; 1.0981x over previous
import jax
import jax.numpy as jnp
from jax import lax
from jax.experimental import pallas as pl
from jax.experimental.pallas import tpu as pltpu
from jax.experimental.pallas import tpu_sc as plsc

B = 4096
K = 10
D = 64
NU = 100000
NI = 100000
F = 10
UFEAT = 3207
IFEAT = 2094

NC = 2
NS = 16
NW = NC * NS
NU_PAD = 100352
N_PER_W = NU_PAD // NW
CHUNK = 112
NCHUNK = N_PER_W // CHUNK
UFEAT_PAD = 3208
IFEAT_PAD = 2096

_mesh = plsc.VectorSubcoreMesh(core_axis_name="c", subcore_axis_name="s")
_sc_params = pltpu.CompilerParams(use_tc_tiling_on_sc=False, needs_layout_passes=False)


def _wid():
  return lax.axis_index("s") * NC + lax.axis_index("c")


def _fire(tbl, idx_v, rows_v, sem, nrows):
  for g in range(nrows // 80):
    pltpu.async_copy(tbl.at[idx_v.at[pl.ds(g * 80, 80)]],
                     rows_v.at[pl.ds(g * 80, 80)], sem)


def _drain(tbl, idx_v, rows_v, sem, nrows):
  for g in range(nrows // 80):
    pltpu.make_async_copy(tbl.at[idx_v.at[pl.ds(g * 80, 80)]],
                          rows_v.at[pl.ds(g * 80, 80)], sem).wait()


def _tree(vals):
  while len(vals) > 1:
    nxt = [vals[i] + vals[i + 1] for i in range(0, len(vals) - 1, 2)]
    if len(vals) % 2:
      nxt.append(vals[-1])
    vals = nxt
  return vals[0]


def _reduce(rows_v, out_v, nodes, r, bf16_acc=False):
  def node(u, carry):
    base = u * r
    for h in range(D // 32):
      sl = pl.ds(h * 32, 32)
      if bf16_acc:
        out_v[u, sl] = _tree([rows_v[base + j, sl] for j in range(r)])
        continue
      acc_a = None
      acc_b = None
      for j0 in range(0, r, 8):
        terms = [plsc.unpack(rows_v[base + j, sl],
                             format=plsc.PackFormat.INTERLEAVED)
                 for j in range(j0, min(j0 + 8, r))]
        ta = _tree([t[0] for t in terms])
        tb = _tree([t[1] for t in terms])
        acc_a = ta if acc_a is None else acc_a + ta
        acc_b = tb if acc_b is None else acc_b + tb
      out_v[u, sl] = plsc.pack(acc_a, acc_b,
                               format=plsc.PackFormat.INTERLEAVED)
    return carry

  lax.fori_loop(0, nodes, node, 0)


def _gather_sum_pipeline(tbl, fidx, outp, bufs, *, nchunks, nodes, r,
                         idx0_fn, orow_fn, bf16_acc=False):
  idx_b, rows_b, out_b, sem_b = bufs
  nrows = nodes * r

  def fetch(c, p):
    pltpu.sync_copy(fidx.at[pl.ds(idx0_fn(c), nrows)],
                    idx_b[p].at[pl.ds(0, nrows)])
    _fire(tbl, idx_b[p], rows_b[p], sem_b[p], nrows)

  def consume(c, p):
    _drain(tbl, idx_b[p], rows_b[p], sem_b[p], nrows)
    _reduce(rows_b[p], out_b[p], nodes, r, bf16_acc)
    pltpu.sync_copy(out_b[p].at[pl.ds(0, nodes)],
                    outp.at[pl.ds(orow_fn(c), nodes)])

  fetch(0, 0)

  def pair(i, carry):
    c0 = 2 * i
    fetch(c0 + 1, 1)
    consume(c0, 0)

    @pl.when(c0 + 2 < nchunks)
    def _():
      fetch(c0 + 2, 0)

    consume(c0 + 1, 1)
    return carry

  lax.fori_loop(0, nchunks // 2, pair, 0)


def _bag_body(pu, pi, uidx, iidx, bagu, bagi, tblu_s, tbli_s,
              idx_a, idx_bb, rows_a, rows_bb, out_a, out_bb, sem_a, sem_bb):
  wid = _wid()
  bufs = ((idx_a, idx_bb), (rows_a, rows_bb), (out_a, out_bb), (sem_a, sem_bb))

  @pl.when(lax.axis_index("s") == 0)
  def _():
    pltpu.sync_copy(pu, tblu_s)
    pltpu.sync_copy(pi, tbli_s)

  plsc.subcore_barrier()

  def run(tbl, fidx, outp):
    _gather_sum_pipeline(
        tbl, fidx, outp, bufs, nchunks=NCHUNK, nodes=CHUNK, r=F,
        idx0_fn=lambda c: (wid * N_PER_W + c * CHUNK) * F,
        orow_fn=lambda c: wid * N_PER_W + c * CHUNK, bf16_acc=True)

  run(tblu_s, uidx, bagu)
  run(tbli_s, iidx, bagi)


_bag_call = pl.kernel(
    _bag_body,
    out_type=(jax.ShapeDtypeStruct((NU_PAD, D), jnp.bfloat16),
              jax.ShapeDtypeStruct((NU_PAD, D), jnp.bfloat16)),
    mesh=_mesh,
    compiler_params=_sc_params,
    scratch_types=[
        pltpu.VMEM_SHARED((UFEAT_PAD, D), jnp.bfloat16),
        pltpu.VMEM_SHARED((IFEAT_PAD, D), jnp.bfloat16),
        pltpu.VMEM((CHUNK * F,), jnp.int32),
        pltpu.VMEM((CHUNK * F,), jnp.int32),
        pltpu.VMEM((CHUNK * F, D), jnp.bfloat16),
        pltpu.VMEM((CHUNK * F, D), jnp.bfloat16),
        pltpu.VMEM((CHUNK, D), jnp.bfloat16),
        pltpu.VMEM((CHUNK, D), jnp.bfloat16),
        pltpu.SemaphoreType.DMA,
        pltpu.SemaphoreType.DMA,
    ],
)


def _agg_body(bagu, bagi, n0r, n1r, n2r, h0s, g1s, t2s,
              idx_a, idx_bb, rows_a, rows_bb, out_a, out_bb, sem_a, sem_bb):
  wid = _wid()
  bufs = ((idx_a, idx_bb), (rows_a, rows_bb), (out_a, out_bb), (sem_a, sem_bb))

  _gather_sum_pipeline(
      bagu, n2r, t2s, bufs, nchunks=16, nodes=8, r=100,
      idx0_fn=lambda c: (wid * 128 + c * 8) * 100,
      orow_fn=lambda c: wid * 128 + c * 8)

  _gather_sum_pipeline(
      bagi, n1r, g1s, bufs, nchunks=8, nodes=16, r=K,
      idx0_fn=lambda c: (wid * 128 + c * 16) * K,
      orow_fn=lambda c: wid * 128 + c * 16)

  pltpu.sync_copy(n0r.at[pl.ds(wid * 128, 128)], idx_a.at[pl.ds(0, 128)])
  for g in range(2):
    pltpu.async_copy(bagu.at[idx_a.at[pl.ds(g * 64, 64)]],
                     rows_a.at[pl.ds(g * 64, 64)], sem_a)
  for g in range(2):
    pltpu.make_async_copy(bagu.at[idx_a.at[pl.ds(g * 64, 64)]],
                          rows_a.at[pl.ds(g * 64, 64)], sem_a).wait()
  pltpu.sync_copy(rows_a.at[pl.ds(0, 128)], h0s.at[pl.ds(wid * 128, 128)])


_agg_call = pl.kernel(
    _agg_body,
    out_type=(jax.ShapeDtypeStruct((B, D), jnp.bfloat16),
              jax.ShapeDtypeStruct((B, D), jnp.bfloat16),
              jax.ShapeDtypeStruct((B, D), jnp.bfloat16)),
    mesh=_mesh,
    compiler_params=_sc_params,
    scratch_types=[
        pltpu.VMEM((800,), jnp.int32),
        pltpu.VMEM((800,), jnp.int32),
        pltpu.VMEM((800, D), jnp.bfloat16),
        pltpu.VMEM((800, D), jnp.bfloat16),
        pltpu.VMEM((16, D), jnp.bfloat16),
        pltpu.VMEM((16, D), jnp.bfloat16),
        pltpu.SemaphoreType.DMA,
        pltpu.SemaphoreType.DMA,
    ],
)


def _dg(a, b):
  return lax.dot_general(a, b, (((1,), (1,)), ((), ())),
                         preferred_element_type=jnp.float32)


def _proj_body(eu_ref, wu_ref, ei_ref, wi_ref, ou_ref, oi_ref):
  ou_ref[...] = (_dg(eu_ref[...], wu_ref[...]) * (1.0 / F)).astype(jnp.bfloat16)
  oi_ref[...] = (_dg(ei_ref[...], wi_ref[...]) * (1.0 / F)).astype(jnp.bfloat16)


def _proj2(eu, wu, ei, wi):
  eu = jnp.pad(eu, ((0, UFEAT_PAD - UFEAT), (0, 0)))
  ei = jnp.pad(ei, ((0, IFEAT_PAD - IFEAT), (0, 0)))
  return pl.pallas_call(
      _proj_body,
      out_shape=(jax.ShapeDtypeStruct((UFEAT_PAD, D), jnp.bfloat16),
                 jax.ShapeDtypeStruct((IFEAT_PAD, D), jnp.bfloat16)),
  )(eu, wu, ei, wi)


def _final_body(h0_ref, g1_ref, t2_ref, w0_ref, w1_ref,
                bu_ref, bi_ref, b0_ref, b1_ref, o_ref):
  h0 = h0_ref[...].astype(jnp.float32) + bu_ref[...]
  g1 = g1_ref[...].astype(jnp.float32) + float(K) * bi_ref[...]
  t2 = t2_ref[...].astype(jnp.float32) + float(K * K) * bu_ref[...]
  w0 = w0_ref[...]
  w1 = w1_ref[...]
  w0a, w0b = w0[:, :D], w0[:, D:]
  w1a, w1b = w1[:, :D], w1[:, D:]
  y0 = _dg(h0, w0a) + _dg(g1, w0b) + b0_ref[...]
  z = _dg(g1, w0a) + _dg(t2, w0b) + float(K) * b0_ref[...]
  o_ref[...] = _dg(y0, w1a) + _dg(z, w1b) + b1_ref[...]


_final = pl.pallas_call(
    _final_body,
    out_shape=jax.ShapeDtypeStruct((B, D), jnp.float32),
)


@jax.jit
def kernel(n0, n1, n2, user_feat_idx, item_feat_idx, user_feat_emb,
           item_feat_emb, user_proj_w, user_proj_b, item_proj_w, item_proj_b,
           w0_w, w0_b, w1_w, w1_b):
  pu, pi = _proj2(user_feat_emb, user_proj_w, item_feat_emb, item_proj_w)
  uidx = jnp.pad(user_feat_idx, (0, (NU_PAD - NU) * F))
  iidx = jnp.pad(item_feat_idx, (0, (NU_PAD - NI) * F))
  bagu, bagi = _bag_call(pu, pi, uidx, iidx)
  h0s, g1s, t2s = _agg_call(bagu, bagi, n0, n1, n2)
  return _final(h0s, g1s, t2s, w0_w, w1_w,
                user_proj_b.reshape(1, D), item_proj_b.reshape(1, D),
                w0_b.reshape(1, D), w1_b.reshape(1, D))


# speedup vs baseline: 6.6111x; 1.0103x over previous
import jax
import jax.numpy as jnp
from jax import lax
from jax.experimental import pallas as pl
from jax.experimental.pallas import tpu as pltpu
from jax.experimental.pallas import tpu_sc as plsc

B = 4096
K = 10
D = 64
NU = 100000
NI = 100000
F = 10
UFEAT = 3207
IFEAT = 2094

NC = 2
NS = 16
NW = NC * NS
NU_PAD = 100352
N_PER_W = NU_PAD // NW
CHUNK = 112
NCHUNK = N_PER_W // CHUNK
UFEAT_PAD = 3208
IFEAT_PAD = 2096

_mesh = plsc.VectorSubcoreMesh(core_axis_name="c", subcore_axis_name="s")
_sc_params = pltpu.CompilerParams(use_tc_tiling_on_sc=False, needs_layout_passes=False)


def _wid():
  return lax.axis_index("s") * NC + lax.axis_index("c")


def _fire(tbl, idx_v, rows_v, sem, nrows):
  for g in range(nrows // 80):
    pltpu.async_copy(tbl.at[idx_v.at[pl.ds(g * 80, 80)]],
                     rows_v.at[pl.ds(g * 80, 80)], sem)


def _drain(tbl, idx_v, rows_v, sem, nrows):
  for g in range(nrows // 80):
    pltpu.make_async_copy(tbl.at[idx_v.at[pl.ds(g * 80, 80)]],
                          rows_v.at[pl.ds(g * 80, 80)], sem).wait()


def _tree(vals):
  while len(vals) > 1:
    nxt = [vals[i] + vals[i + 1] for i in range(0, len(vals) - 1, 2)]
    if len(vals) % 2:
      nxt.append(vals[-1])
    vals = nxt
  return vals[0]


def _reduce(rows_v, out_v, nodes, r, bf16_acc=False):
  def node(u, carry):
    base = u * r
    for h in range(D // 32):
      sl = pl.ds(h * 32, 32)
      if bf16_acc:
        out_v[u, sl] = _tree([rows_v[base + j, sl] for j in range(r)])
        continue
      if r >= 20:
        parts = []
        for j0 in range(0, r, 10):
          pb = _tree([rows_v[base + j, sl] for j in range(j0, j0 + 10)])
          parts.append(plsc.unpack(pb, format=plsc.PackFormat.INTERLEAVED))
        pa = _tree([x[0] for x in parts])
        pbb = _tree([x[1] for x in parts])
        out_v[u, sl] = plsc.pack(pa, pbb, format=plsc.PackFormat.INTERLEAVED)
        continue
      acc_a = None
      acc_b = None
      for j0 in range(0, r, 8):
        terms = [plsc.unpack(rows_v[base + j, sl],
                             format=plsc.PackFormat.INTERLEAVED)
                 for j in range(j0, min(j0 + 8, r))]
        ta = _tree([t[0] for t in terms])
        tb = _tree([t[1] for t in terms])
        acc_a = ta if acc_a is None else acc_a + ta
        acc_b = tb if acc_b is None else acc_b + tb
      out_v[u, sl] = plsc.pack(acc_a, acc_b,
                               format=plsc.PackFormat.INTERLEAVED)
    return carry

  lax.fori_loop(0, nodes, node, 0)


def _gather_sum_pipeline(tbl, fidx, outp, bufs, *, nchunks, nodes, r,
                         idx0_fn, orow_fn, bf16_acc=False):
  idx_b, rows_b, out_b, sem_b = bufs
  nrows = nodes * r

  def fetch(c, p):
    pltpu.sync_copy(fidx.at[pl.ds(idx0_fn(c), nrows)],
                    idx_b[p].at[pl.ds(0, nrows)])
    _fire(tbl, idx_b[p], rows_b[p], sem_b[p], nrows)

  def consume(c, p):
    _drain(tbl, idx_b[p], rows_b[p], sem_b[p], nrows)
    _reduce(rows_b[p], out_b[p], nodes, r, bf16_acc)
    pltpu.sync_copy(out_b[p].at[pl.ds(0, nodes)],
                    outp.at[pl.ds(orow_fn(c), nodes)])

  fetch(0, 0)

  def pair(i, carry):
    c0 = 2 * i
    fetch(c0 + 1, 1)
    consume(c0, 0)

    @pl.when(c0 + 2 < nchunks)
    def _():
      fetch(c0 + 2, 0)

    consume(c0 + 1, 1)
    return carry

  lax.fori_loop(0, nchunks // 2, pair, 0)


def _bag_body(pu, pi, uidx, iidx, bagu, bagi, tblu_s, tbli_s,
              idx_a, idx_bb, rows_a, rows_bb, out_a, out_bb, sem_a, sem_bb):
  wid = _wid()
  bufs = ((idx_a, idx_bb), (rows_a, rows_bb), (out_a, out_bb), (sem_a, sem_bb))

  @pl.when(lax.axis_index("s") == 0)
  def _():
    pltpu.sync_copy(pu, tblu_s)
    pltpu.sync_copy(pi, tbli_s)

  plsc.subcore_barrier()

  def run(tbl, fidx, outp):
    _gather_sum_pipeline(
        tbl, fidx, outp, bufs, nchunks=NCHUNK, nodes=CHUNK, r=F,
        idx0_fn=lambda c: (wid * N_PER_W + c * CHUNK) * F,
        orow_fn=lambda c: wid * N_PER_W + c * CHUNK, bf16_acc=True)

  run(tblu_s, uidx, bagu)
  run(tbli_s, iidx, bagi)


_bag_call = pl.kernel(
    _bag_body,
    out_type=(jax.ShapeDtypeStruct((NU_PAD, D), jnp.bfloat16),
              jax.ShapeDtypeStruct((NU_PAD, D), jnp.bfloat16)),
    mesh=_mesh,
    compiler_params=_sc_params,
    scratch_types=[
        pltpu.VMEM_SHARED((UFEAT_PAD, D), jnp.bfloat16),
        pltpu.VMEM_SHARED((IFEAT_PAD, D), jnp.bfloat16),
        pltpu.VMEM((CHUNK * F,), jnp.int32),
        pltpu.VMEM((CHUNK * F,), jnp.int32),
        pltpu.VMEM((CHUNK * F, D), jnp.bfloat16),
        pltpu.VMEM((CHUNK * F, D), jnp.bfloat16),
        pltpu.VMEM((CHUNK, D), jnp.bfloat16),
        pltpu.VMEM((CHUNK, D), jnp.bfloat16),
        pltpu.SemaphoreType.DMA,
        pltpu.SemaphoreType.DMA,
    ],
)


def _agg_body(bagu, bagi, n0r, n1r, n2r, h0s, g1s, t2s,
              idx_a, idx_bb, rows_a, rows_bb, out_a, out_bb, sem_a, sem_bb):
  wid = _wid()
  bufs = ((idx_a, idx_bb), (rows_a, rows_bb), (out_a, out_bb), (sem_a, sem_bb))

  _gather_sum_pipeline(
      bagu, n2r, t2s, bufs, nchunks=16, nodes=8, r=100,
      idx0_fn=lambda c: (wid * 128 + c * 8) * 100,
      orow_fn=lambda c: wid * 128 + c * 8)

  _gather_sum_pipeline(
      bagi, n1r, g1s, bufs, nchunks=8, nodes=16, r=K,
      idx0_fn=lambda c: (wid * 128 + c * 16) * K,
      orow_fn=lambda c: wid * 128 + c * 16)

  pltpu.sync_copy(n0r.at[pl.ds(wid * 128, 128)], idx_a.at[pl.ds(0, 128)])
  for g in range(2):
    pltpu.async_copy(bagu.at[idx_a.at[pl.ds(g * 64, 64)]],
                     rows_a.at[pl.ds(g * 64, 64)], sem_a)
  for g in range(2):
    pltpu.make_async_copy(bagu.at[idx_a.at[pl.ds(g * 64, 64)]],
                          rows_a.at[pl.ds(g * 64, 64)], sem_a).wait()
  pltpu.sync_copy(rows_a.at[pl.ds(0, 128)], h0s.at[pl.ds(wid * 128, 128)])


_agg_call = pl.kernel(
    _agg_body,
    out_type=(jax.ShapeDtypeStruct((B, D), jnp.bfloat16),
              jax.ShapeDtypeStruct((B, D), jnp.bfloat16),
              jax.ShapeDtypeStruct((B, D), jnp.bfloat16)),
    mesh=_mesh,
    compiler_params=_sc_params,
    scratch_types=[
        pltpu.VMEM((800,), jnp.int32),
        pltpu.VMEM((800,), jnp.int32),
        pltpu.VMEM((800, D), jnp.bfloat16),
        pltpu.VMEM((800, D), jnp.bfloat16),
        pltpu.VMEM((16, D), jnp.bfloat16),
        pltpu.VMEM((16, D), jnp.bfloat16),
        pltpu.SemaphoreType.DMA,
        pltpu.SemaphoreType.DMA,
    ],
)


def _dg(a, b):
  return lax.dot_general(a, b, (((1,), (1,)), ((), ())),
                         preferred_element_type=jnp.float32)


def _proj_body(eu_ref, wu_ref, ei_ref, wi_ref, ou_ref, oi_ref):
  ou_ref[...] = (_dg(eu_ref[...], wu_ref[...]) * (1.0 / F)).astype(jnp.bfloat16)
  oi_ref[...] = (_dg(ei_ref[...], wi_ref[...]) * (1.0 / F)).astype(jnp.bfloat16)


def _proj2(eu, wu, ei, wi):
  eu = jnp.pad(eu, ((0, UFEAT_PAD - UFEAT), (0, 0)))
  ei = jnp.pad(ei, ((0, IFEAT_PAD - IFEAT), (0, 0)))
  return pl.pallas_call(
      _proj_body,
      out_shape=(jax.ShapeDtypeStruct((UFEAT_PAD, D), jnp.bfloat16),
                 jax.ShapeDtypeStruct((IFEAT_PAD, D), jnp.bfloat16)),
  )(eu, wu, ei, wi)


def _final_body(h0_ref, g1_ref, t2_ref, w0_ref, w1_ref,
                bu_ref, bi_ref, b0_ref, b1_ref, o_ref):
  h0 = h0_ref[...].astype(jnp.float32) + bu_ref[...]
  g1 = g1_ref[...].astype(jnp.float32) + float(K) * bi_ref[...]
  t2 = t2_ref[...].astype(jnp.float32) + float(K * K) * bu_ref[...]
  w0 = w0_ref[...]
  w1 = w1_ref[...]
  w0a, w0b = w0[:, :D], w0[:, D:]
  w1a, w1b = w1[:, :D], w1[:, D:]
  y0 = _dg(h0, w0a) + _dg(g1, w0b) + b0_ref[...]
  z = _dg(g1, w0a) + _dg(t2, w0b) + float(K) * b0_ref[...]
  o_ref[...] = _dg(y0, w1a) + _dg(z, w1b) + b1_ref[...]


_final = pl.pallas_call(
    _final_body,
    out_shape=jax.ShapeDtypeStruct((B, D), jnp.float32),
)


@jax.jit
def kernel(n0, n1, n2, user_feat_idx, item_feat_idx, user_feat_emb,
           item_feat_emb, user_proj_w, user_proj_b, item_proj_w, item_proj_b,
           w0_w, w0_b, w1_w, w1_b):
  pu, pi = _proj2(user_feat_emb, user_proj_w, item_feat_emb, item_proj_w)
  uidx = jnp.pad(user_feat_idx, (0, (NU_PAD - NU) * F))
  iidx = jnp.pad(item_feat_idx, (0, (NU_PAD - NI) * F))
  bagu, bagi = _bag_call(pu, pi, uidx, iidx)
  h0s, g1s, t2s = _agg_call(bagu, bagi, n0, n1, n2)
  return _final(h0s, g1s, t2s, w0_w, w1_w,
                user_proj_b.reshape(1, D), item_proj_b.reshape(1, D),
                w0_b.reshape(1, D), w1_b.reshape(1, D))


# speedup vs baseline: 7.4250x; 1.1231x over previous
import jax
import jax.numpy as jnp
from jax import lax
from jax.experimental import pallas as pl
from jax.experimental.pallas import tpu as pltpu
from jax.experimental.pallas import tpu_sc as plsc

B = 4096
K = 10
D = 64
NU = 100000
NI = 100000
F = 10
UFEAT = 3207
IFEAT = 2094

NC = 2
NS = 16
NW = NC * NS
NU_PAD = 100352
N_PER_W = NU_PAD // NW
CHUNK = 56
NCHUNK = N_PER_W // CHUNK
UFEAT_PAD = 3208
IFEAT_PAD = 2096

_mesh = plsc.VectorSubcoreMesh(core_axis_name="c", subcore_axis_name="s")
_sc_params = pltpu.CompilerParams(use_tc_tiling_on_sc=False, needs_layout_passes=False)


def _wid():
  return lax.axis_index("s") * NC + lax.axis_index("c")


def _fire(tbl, idx_v, coff, rows_v, sem, nrows):
  for g in range(nrows // 80):
    pltpu.async_copy(tbl.at[idx_v.at[pl.ds(coff + g * 80, 80)]],
                     rows_v.at[pl.ds(g * 80, 80)], sem)


def _drain(tbl, idx_v, rows_v, sem, nrows):
  for g in range(nrows // 80):
    pltpu.make_async_copy(tbl.at[idx_v.at[pl.ds(g * 80, 80)]],
                          rows_v.at[pl.ds(g * 80, 80)], sem).wait()


def _tree(vals):
  while len(vals) > 1:
    nxt = [vals[i] + vals[i + 1] for i in range(0, len(vals) - 1, 2)]
    if len(vals) % 2:
      nxt.append(vals[-1])
    vals = nxt
  return vals[0]


def _reduce(rows_v, out_v, nodes, r, bf16_acc=False):
  def node(u, carry):
    base = u * r
    for h in range(D // 32):
      sl = pl.ds(h * 32, 32)
      if bf16_acc:
        out_v[u, sl] = _tree([rows_v[base + j, sl] for j in range(r)])
        continue
      if r >= 20:
        parts = []
        for j0 in range(0, r, 10):
          pb = _tree([rows_v[base + j, sl] for j in range(j0, j0 + 10)])
          parts.append(plsc.unpack(pb, format=plsc.PackFormat.INTERLEAVED))
        pa = _tree([x[0] for x in parts])
        pbb = _tree([x[1] for x in parts])
        out_v[u, sl] = plsc.pack(pa, pbb, format=plsc.PackFormat.INTERLEAVED)
        continue
      acc_a = None
      acc_b = None
      for j0 in range(0, r, 8):
        terms = [plsc.unpack(rows_v[base + j, sl],
                             format=plsc.PackFormat.INTERLEAVED)
                 for j in range(j0, min(j0 + 8, r))]
        ta = _tree([t[0] for t in terms])
        tb = _tree([t[1] for t in terms])
        acc_a = ta if acc_a is None else acc_a + ta
        acc_b = tb if acc_b is None else acc_b + tb
      out_v[u, sl] = plsc.pack(acc_a, acc_b,
                               format=plsc.PackFormat.INTERLEAVED)
    return carry

  lax.fori_loop(0, nodes, node, 0)


def _gather_sum_pipeline(tbl, fidx, outp, bufs, *, nchunks, nodes, r,
                         idx0, orow0, bf16_acc=False):
  idxall, rows_b, out_b, sem_b = bufs
  nrows = nodes * r

  pltpu.sync_copy(fidx.at[pl.ds(idx0, nchunks * nrows)],
                  idxall.at[pl.ds(0, nchunks * nrows)])

  def fetch(c, p):
    _fire(tbl, idxall, c * nrows, rows_b[p], sem_b[p], nrows)

  def consume(c, p):
    _drain(tbl, idxall, rows_b[p], sem_b[p], nrows)
    _reduce(rows_b[p], out_b[p], nodes, r, bf16_acc)
    pltpu.sync_copy(out_b[p].at[pl.ds(0, nodes)],
                    outp.at[pl.ds(orow0 + c * nodes, nodes)])

  fetch(0, 0)

  def pair(i, carry):
    c0 = 2 * i
    fetch(c0 + 1, 1)
    consume(c0, 0)

    @pl.when(c0 + 2 < nchunks)
    def _():
      fetch(c0 + 2, 0)

    consume(c0 + 1, 1)
    return carry

  lax.fori_loop(0, nchunks // 2, pair, 0)


def _bag_body(pu, pi, uidx, iidx, bagu, bagi, tblu_s, tbli_s,
              idxall, rows_a, rows_bb, out_a, out_bb, sem_a, sem_bb):
  wid = _wid()
  bufs = (idxall, (rows_a, rows_bb), (out_a, out_bb), (sem_a, sem_bb))

  @pl.when(lax.axis_index("s") == 0)
  def _():
    pltpu.sync_copy(pu, tblu_s)
    pltpu.sync_copy(pi, tbli_s)

  plsc.subcore_barrier()

  def run(tbl, fidx, outp):
    _gather_sum_pipeline(
        tbl, fidx, outp, bufs, nchunks=NCHUNK, nodes=CHUNK, r=F,
        idx0=wid * N_PER_W * F, orow0=wid * N_PER_W, bf16_acc=True)

  run(tblu_s, uidx, bagu)
  run(tbli_s, iidx, bagi)


_bag_call = pl.kernel(
    _bag_body,
    out_type=(jax.ShapeDtypeStruct((NU_PAD, D), jnp.bfloat16),
              jax.ShapeDtypeStruct((NU_PAD, D), jnp.bfloat16)),
    mesh=_mesh,
    compiler_params=_sc_params,
    scratch_types=[
        pltpu.VMEM_SHARED((UFEAT_PAD, D), jnp.bfloat16),
        pltpu.VMEM_SHARED((IFEAT_PAD, D), jnp.bfloat16),
        pltpu.VMEM((N_PER_W * F,), jnp.int32),
        pltpu.VMEM((CHUNK * F, D), jnp.bfloat16),
        pltpu.VMEM((CHUNK * F, D), jnp.bfloat16),
        pltpu.VMEM((CHUNK, D), jnp.bfloat16),
        pltpu.VMEM((CHUNK, D), jnp.bfloat16),
        pltpu.SemaphoreType.DMA,
        pltpu.SemaphoreType.DMA,
    ],
)


def _agg_body(bagu, bagi, n0r, n1r, n2r, h0s, g1s, t2s,
              idxall, rows_a, rows_bb, out_a, out_bb, sem_a, sem_bb):
  wid = _wid()
  bufs = (idxall, (rows_a, rows_bb), (out_a, out_bb), (sem_a, sem_bb))

  _gather_sum_pipeline(
      bagu, n2r, t2s, bufs, nchunks=16, nodes=8, r=100,
      idx0=wid * 128 * 100, orow0=wid * 128)

  _gather_sum_pipeline(
      bagi, n1r, g1s, bufs, nchunks=8, nodes=16, r=K,
      idx0=wid * 128 * K, orow0=wid * 128)

  pltpu.sync_copy(n0r.at[pl.ds(wid * 128, 128)], idxall.at[pl.ds(0, 128)])
  for g in range(2):
    pltpu.async_copy(bagu.at[idxall.at[pl.ds(g * 64, 64)]],
                     rows_a.at[pl.ds(g * 64, 64)], sem_a)
  for g in range(2):
    pltpu.make_async_copy(bagu.at[idxall.at[pl.ds(g * 64, 64)]],
                          rows_a.at[pl.ds(g * 64, 64)], sem_a).wait()
  pltpu.sync_copy(rows_a.at[pl.ds(0, 128)], h0s.at[pl.ds(wid * 128, 128)])


_agg_call = pl.kernel(
    _agg_body,
    out_type=(jax.ShapeDtypeStruct((B, D), jnp.bfloat16),
              jax.ShapeDtypeStruct((B, D), jnp.bfloat16),
              jax.ShapeDtypeStruct((B, D), jnp.bfloat16)),
    mesh=_mesh,
    compiler_params=_sc_params,
    scratch_types=[
        pltpu.VMEM((12800,), jnp.int32),
        pltpu.VMEM((800, D), jnp.bfloat16),
        pltpu.VMEM((800, D), jnp.bfloat16),
        pltpu.VMEM((16, D), jnp.bfloat16),
        pltpu.VMEM((16, D), jnp.bfloat16),
        pltpu.SemaphoreType.DMA,
        pltpu.SemaphoreType.DMA,
    ],
)


def _dg(a, b):
  return lax.dot_general(a, b, (((1,), (1,)), ((), ())),
                         preferred_element_type=jnp.float32)


def _proj_body(eu_ref, wu_ref, ei_ref, wi_ref, ou_ref, oi_ref):
  ou_ref[...] = (_dg(eu_ref[...], wu_ref[...]) * (1.0 / F)).astype(jnp.bfloat16)
  oi_ref[...] = (_dg(ei_ref[...], wi_ref[...]) * (1.0 / F)).astype(jnp.bfloat16)


def _proj2(eu, wu, ei, wi):
  eu = jnp.pad(eu, ((0, UFEAT_PAD - UFEAT), (0, 0)))
  ei = jnp.pad(ei, ((0, IFEAT_PAD - IFEAT), (0, 0)))
  return pl.pallas_call(
      _proj_body,
      out_shape=(jax.ShapeDtypeStruct((UFEAT_PAD, D), jnp.bfloat16),
                 jax.ShapeDtypeStruct((IFEAT_PAD, D), jnp.bfloat16)),
  )(eu, wu, ei, wi)


def _final_body(h0_ref, g1_ref, t2_ref, w0_ref, w1_ref,
                bu_ref, bi_ref, b0_ref, b1_ref, o_ref):
  h0 = h0_ref[...].astype(jnp.float32) + bu_ref[...]
  g1 = g1_ref[...].astype(jnp.float32) + float(K) * bi_ref[...]
  t2 = t2_ref[...].astype(jnp.float32) + float(K * K) * bu_ref[...]
  w0 = w0_ref[...]
  w1 = w1_ref[...]
  w0a, w0b = w0[:, :D], w0[:, D:]
  w1a, w1b = w1[:, :D], w1[:, D:]
  y0 = _dg(h0, w0a) + _dg(g1, w0b) + b0_ref[...]
  z = _dg(g1, w0a) + _dg(t2, w0b) + float(K) * b0_ref[...]
  o_ref[...] = _dg(y0, w1a) + _dg(z, w1b) + b1_ref[...]


_final = pl.pallas_call(
    _final_body,
    out_shape=jax.ShapeDtypeStruct((B, D), jnp.float32),
)


@jax.jit
def kernel(n0, n1, n2, user_feat_idx, item_feat_idx, user_feat_emb,
           item_feat_emb, user_proj_w, user_proj_b, item_proj_w, item_proj_b,
           w0_w, w0_b, w1_w, w1_b):
  pu, pi = _proj2(user_feat_emb, user_proj_w, item_feat_emb, item_proj_w)
  uidx = jnp.pad(user_feat_idx, (0, (NU_PAD - NU) * F))
  iidx = jnp.pad(item_feat_idx, (0, (NU_PAD - NI) * F))
  bagu, bagi = _bag_call(pu, pi, uidx, iidx)
  h0s, g1s, t2s = _agg_call(bagu, bagi, n0, n1, n2)
  return _final(h0s, g1s, t2s, w0_w, w1_w,
                user_proj_b.reshape(1, D), item_proj_b.reshape(1, D),
                w0_b.reshape(1, D), w1_b.reshape(1, D))


# speedup vs baseline: 7.4511x; 1.0035x over previous
import jax
import jax.numpy as jnp
from jax import lax
from jax.experimental import pallas as pl
from jax.experimental.pallas import tpu as pltpu
from jax.experimental.pallas import tpu_sc as plsc

B = 4096
K = 10
D = 64
NU = 100000
NI = 100000
F = 10
UFEAT = 3207
IFEAT = 2094

NC = 2
NS = 16
NW = NC * NS
NU_PAD = 100352
N_PER_W = NU_PAD // NW
CHUNK = 112
NCHUNK = N_PER_W // CHUNK
UFEAT_PAD = 3208
IFEAT_PAD = 2096

_mesh = plsc.VectorSubcoreMesh(core_axis_name="c", subcore_axis_name="s")
_sc_params = pltpu.CompilerParams(use_tc_tiling_on_sc=False, needs_layout_passes=False)


def _wid():
  return lax.axis_index("s") * NC + lax.axis_index("c")


def _fire(tbl, idx_v, coff, rows_v, sem, nrows):
  for g in range(nrows // 80):
    pltpu.async_copy(tbl.at[idx_v.at[pl.ds(coff + g * 80, 80)]],
                     rows_v.at[pl.ds(g * 80, 80)], sem)


def _drain(tbl, idx_v, rows_v, sem, nrows):
  for g in range(nrows // 80):
    pltpu.make_async_copy(tbl.at[idx_v.at[pl.ds(g * 80, 80)]],
                          rows_v.at[pl.ds(g * 80, 80)], sem).wait()


def _tree(vals):
  while len(vals) > 1:
    nxt = [vals[i] + vals[i + 1] for i in range(0, len(vals) - 1, 2)]
    if len(vals) % 2:
      nxt.append(vals[-1])
    vals = nxt
  return vals[0]


def _reduce(rows_v, out_v, nodes, r, bf16_acc=False):
  def node(u, carry):
    base = u * r
    for h in range(D // 32):
      sl = pl.ds(h * 32, 32)
      if bf16_acc:
        out_v[u, sl] = _tree([rows_v[base + j, sl] for j in range(r)])
        continue
      if r >= 20:
        parts = []
        for j0 in range(0, r, 10):
          pb = _tree([rows_v[base + j, sl] for j in range(j0, j0 + 10)])
          parts.append(plsc.unpack(pb, format=plsc.PackFormat.INTERLEAVED))
        pa = _tree([x[0] for x in parts])
        pbb = _tree([x[1] for x in parts])
        out_v[u, sl] = plsc.pack(pa, pbb, format=plsc.PackFormat.INTERLEAVED)
        continue
      acc_a = None
      acc_b = None
      for j0 in range(0, r, 8):
        terms = [plsc.unpack(rows_v[base + j, sl],
                             format=plsc.PackFormat.INTERLEAVED)
                 for j in range(j0, min(j0 + 8, r))]
        ta = _tree([t[0] for t in terms])
        tb = _tree([t[1] for t in terms])
        acc_a = ta if acc_a is None else acc_a + ta
        acc_b = tb if acc_b is None else acc_b + tb
      out_v[u, sl] = plsc.pack(acc_a, acc_b,
                               format=plsc.PackFormat.INTERLEAVED)
    return carry

  lax.fori_loop(0, nodes, node, 0)


def _gather_sum_pipeline(tbl, fidx, outp, bufs, *, nchunks, nodes, r,
                         idx0, orow0, bf16_acc=False):
  idxall, rows_b, out_b, sem_b = bufs
  nrows = nodes * r

  pltpu.sync_copy(fidx.at[pl.ds(idx0, nchunks * nrows)],
                  idxall.at[pl.ds(0, nchunks * nrows)])

  def fetch(c, p):
    _fire(tbl, idxall, c * nrows, rows_b[p], sem_b[p], nrows)

  def consume(c, p):
    _drain(tbl, idxall, rows_b[p], sem_b[p], nrows)
    _reduce(rows_b[p], out_b[p], nodes, r, bf16_acc)
    pltpu.sync_copy(out_b[p].at[pl.ds(0, nodes)],
                    outp.at[pl.ds(orow0 + c * nodes, nodes)])

  fetch(0, 0)

  def pair(i, carry):
    c0 = 2 * i
    fetch(c0 + 1, 1)
    consume(c0, 0)

    @pl.when(c0 + 2 < nchunks)
    def _():
      fetch(c0 + 2, 0)

    consume(c0 + 1, 1)
    return carry

  lax.fori_loop(0, nchunks // 2, pair, 0)


def _bag_body(pu, pi, uidx, iidx, bagu, bagi, tblu_s, tbli_s,
              idxall, rows_a, rows_bb, out_a, out_bb, sem_a, sem_bb):
  wid = _wid()
  bufs = (idxall, (rows_a, rows_bb), (out_a, out_bb), (sem_a, sem_bb))

  @pl.when(lax.axis_index("s") == 0)
  def _():
    pltpu.sync_copy(pu, tblu_s)
    pltpu.sync_copy(pi, tbli_s)

  plsc.subcore_barrier()

  def run(tbl, fidx, outp):
    _gather_sum_pipeline(
        tbl, fidx, outp, bufs, nchunks=NCHUNK, nodes=CHUNK, r=F,
        idx0=wid * N_PER_W * F, orow0=wid * N_PER_W, bf16_acc=True)

  run(tblu_s, uidx, bagu)
  run(tbli_s, iidx, bagi)


_bag_call = pl.kernel(
    _bag_body,
    out_type=(jax.ShapeDtypeStruct((NU_PAD, D), jnp.bfloat16),
              jax.ShapeDtypeStruct((NU_PAD, D), jnp.bfloat16)),
    mesh=_mesh,
    compiler_params=_sc_params,
    scratch_types=[
        pltpu.VMEM_SHARED((UFEAT_PAD, D), jnp.bfloat16),
        pltpu.VMEM_SHARED((IFEAT_PAD, D), jnp.bfloat16),
        pltpu.VMEM((N_PER_W * F,), jnp.int32),
        pltpu.VMEM((CHUNK * F, D), jnp.bfloat16),
        pltpu.VMEM((CHUNK * F, D), jnp.bfloat16),
        pltpu.VMEM((CHUNK, D), jnp.bfloat16),
        pltpu.VMEM((CHUNK, D), jnp.bfloat16),
        pltpu.SemaphoreType.DMA,
        pltpu.SemaphoreType.DMA,
    ],
)


def _agg_body(bagu, bagi, n0r, n1r, n2r, h0s, g1s, t2s,
              idxall, rows_a, rows_bb, out_a, out_bb, sem_a, sem_bb):
  wid = _wid()
  bufs = (idxall, (rows_a, rows_bb), (out_a, out_bb), (sem_a, sem_bb))

  _gather_sum_pipeline(
      bagu, n2r, t2s, bufs, nchunks=16, nodes=8, r=100,
      idx0=wid * 128 * 100, orow0=wid * 128)

  _gather_sum_pipeline(
      bagi, n1r, g1s, bufs, nchunks=8, nodes=16, r=K,
      idx0=wid * 128 * K, orow0=wid * 128)

  pltpu.sync_copy(n0r.at[pl.ds(wid * 128, 128)], idxall.at[pl.ds(0, 128)])
  for g in range(2):
    pltpu.async_copy(bagu.at[idxall.at[pl.ds(g * 64, 64)]],
                     rows_a.at[pl.ds(g * 64, 64)], sem_a)
  for g in range(2):
    pltpu.make_async_copy(bagu.at[idxall.at[pl.ds(g * 64, 64)]],
                          rows_a.at[pl.ds(g * 64, 64)], sem_a).wait()
  pltpu.sync_copy(rows_a.at[pl.ds(0, 128)], h0s.at[pl.ds(wid * 128, 128)])


_agg_call = pl.kernel(
    _agg_body,
    out_type=(jax.ShapeDtypeStruct((B, D), jnp.bfloat16),
              jax.ShapeDtypeStruct((B, D), jnp.bfloat16),
              jax.ShapeDtypeStruct((B, D), jnp.bfloat16)),
    mesh=_mesh,
    compiler_params=_sc_params,
    scratch_types=[
        pltpu.VMEM((12800,), jnp.int32),
        pltpu.VMEM((800, D), jnp.bfloat16),
        pltpu.VMEM((800, D), jnp.bfloat16),
        pltpu.VMEM((16, D), jnp.bfloat16),
        pltpu.VMEM((16, D), jnp.bfloat16),
        pltpu.SemaphoreType.DMA,
        pltpu.SemaphoreType.DMA,
    ],
)


def _dg(a, b):
  return lax.dot_general(a, b, (((1,), (1,)), ((), ())),
                         preferred_element_type=jnp.float32)


def _proj_body(eu_ref, wu_ref, ei_ref, wi_ref, ou_ref, oi_ref):
  ou_ref[...] = (_dg(eu_ref[...], wu_ref[...]) * (1.0 / F)).astype(jnp.bfloat16)
  oi_ref[...] = (_dg(ei_ref[...], wi_ref[...]) * (1.0 / F)).astype(jnp.bfloat16)


def _proj2(eu, wu, ei, wi):
  eu = jnp.pad(eu, ((0, UFEAT_PAD - UFEAT), (0, 0)))
  ei = jnp.pad(ei, ((0, IFEAT_PAD - IFEAT), (0, 0)))
  return pl.pallas_call(
      _proj_body,
      out_shape=(jax.ShapeDtypeStruct((UFEAT_PAD, D), jnp.bfloat16),
                 jax.ShapeDtypeStruct((IFEAT_PAD, D), jnp.bfloat16)),
  )(eu, wu, ei, wi)


def _final_body(h0_ref, g1_ref, t2_ref, w0_ref, w1_ref,
                bu_ref, bi_ref, b0_ref, b1_ref, o_ref):
  h0 = h0_ref[...].astype(jnp.float32) + bu_ref[...]
  g1 = g1_ref[...].astype(jnp.float32) + float(K) * bi_ref[...]
  t2 = t2_ref[...].astype(jnp.float32) + float(K * K) * bu_ref[...]
  w0 = w0_ref[...]
  w1 = w1_ref[...]
  w0a, w0b = w0[:, :D], w0[:, D:]
  w1a, w1b = w1[:, :D], w1[:, D:]
  y0 = _dg(h0, w0a) + _dg(g1, w0b) + b0_ref[...]
  z = _dg(g1, w0a) + _dg(t2, w0b) + float(K) * b0_ref[...]
  o_ref[...] = _dg(y0, w1a) + _dg(z, w1b) + b1_ref[...]


_final = pl.pallas_call(
    _final_body,
    out_shape=jax.ShapeDtypeStruct((B, D), jnp.float32),
)


@jax.jit
def kernel(n0, n1, n2, user_feat_idx, item_feat_idx, user_feat_emb,
           item_feat_emb, user_proj_w, user_proj_b, item_proj_w, item_proj_b,
           w0_w, w0_b, w1_w, w1_b):
  pu, pi = _proj2(user_feat_emb, user_proj_w, item_feat_emb, item_proj_w)
  uidx = jnp.pad(user_feat_idx, (0, (NU_PAD - NU) * F))
  iidx = jnp.pad(item_feat_idx, (0, (NU_PAD - NI) * F))
  bagu, bagi = _bag_call(pu, pi, uidx, iidx)
  h0s, g1s, t2s = _agg_call(bagu, bagi, n0, n1, n2)
  return _final(h0s, g1s, t2s, w0_w, w1_w,
                user_proj_b.reshape(1, D), item_proj_b.reshape(1, D),
                w0_b.reshape(1, D), w1_b.reshape(1, D))


# speedup vs baseline: 7.4557x; 1.0006x over previous
import jax
import jax.numpy as jnp
from jax import lax
from jax.experimental import pallas as pl
from jax.experimental.pallas import tpu as pltpu
from jax.experimental.pallas import tpu_sc as plsc

B = 4096
K = 10
D = 64
NU = 100000
NI = 100000
F = 10
UFEAT = 3207
IFEAT = 2094

NC = 2
NS = 16
NW = NC * NS
NU_PAD = 100352
N_PER_W = NU_PAD // NW
CHUNK = 112
NCHUNK = N_PER_W // CHUNK
UFEAT_PAD = 3208
IFEAT_PAD = 2096

_mesh = plsc.VectorSubcoreMesh(core_axis_name="c", subcore_axis_name="s",
                               num_cores=NC, num_subcores=NS)
_sc_params = pltpu.CompilerParams(use_tc_tiling_on_sc=False, needs_layout_passes=False)


def _wid():
  return lax.axis_index("s") * NC + lax.axis_index("c")


def _fire(tbl, idx_v, coff, rows_v, sem, nrows):
  for g in range(nrows // 80):
    pltpu.async_copy(tbl.at[idx_v.at[pl.ds(coff + g * 80, 80)]],
                     rows_v.at[pl.ds(g * 80, 80)], sem)


def _drain(tbl, idx_v, rows_v, sem, nrows):
  for g in range(nrows // 80):
    pltpu.make_async_copy(tbl.at[idx_v.at[pl.ds(g * 80, 80)]],
                          rows_v.at[pl.ds(g * 80, 80)], sem).wait()


def _tree(vals):
  while len(vals) > 1:
    nxt = [vals[i] + vals[i + 1] for i in range(0, len(vals) - 1, 2)]
    if len(vals) % 2:
      nxt.append(vals[-1])
    vals = nxt
  return vals[0]


def _reduce(rows_v, out_v, nodes, r, bf16_acc=False):
  def node(u, carry):
    base = u * r
    for h in range(D // 32):
      sl = pl.ds(h * 32, 32)
      if bf16_acc:
        out_v[u, sl] = _tree([rows_v[base + j, sl] for j in range(r)])
        continue
      if r >= 20:
        parts = []
        for j0 in range(0, r, 10):
          pb = _tree([rows_v[base + j, sl] for j in range(j0, j0 + 10)])
          parts.append(plsc.unpack(pb, format=plsc.PackFormat.INTERLEAVED))
        pa = _tree([x[0] for x in parts])
        pbb = _tree([x[1] for x in parts])
        out_v[u, sl] = plsc.pack(pa, pbb, format=plsc.PackFormat.INTERLEAVED)
        continue
      acc_a = None
      acc_b = None
      for j0 in range(0, r, 8):
        terms = [plsc.unpack(rows_v[base + j, sl],
                             format=plsc.PackFormat.INTERLEAVED)
                 for j in range(j0, min(j0 + 8, r))]
        ta = _tree([t[0] for t in terms])
        tb = _tree([t[1] for t in terms])
        acc_a = ta if acc_a is None else acc_a + ta
        acc_b = tb if acc_b is None else acc_b + tb
      out_v[u, sl] = plsc.pack(acc_a, acc_b,
                               format=plsc.PackFormat.INTERLEAVED)
    return carry

  lax.fori_loop(0, nodes, node, 0)


def _gather_sum_pipeline(tbl, fidx, outp, bufs, *, nchunks, nodes, r,
                         idx0, orow0, bf16_acc=False):
  idxall, rows_b, out_b, sem_b = bufs
  nrows = nodes * r

  pltpu.sync_copy(fidx.at[pl.ds(idx0, nchunks * nrows)],
                  idxall.at[pl.ds(0, nchunks * nrows)])

  def fetch(c, p):
    _fire(tbl, idxall, c * nrows, rows_b[p], sem_b[p], nrows)

  def consume(c, p):
    _drain(tbl, idxall, rows_b[p], sem_b[p], nrows)
    _reduce(rows_b[p], out_b[p], nodes, r, bf16_acc)
    pltpu.sync_copy(out_b[p].at[pl.ds(0, nodes)],
                    outp.at[pl.ds(orow0 + c * nodes, nodes)])

  fetch(0, 0)

  def pair(i, carry):
    c0 = 2 * i
    fetch(c0 + 1, 1)
    consume(c0, 0)

    @pl.when(c0 + 2 < nchunks)
    def _():
      fetch(c0 + 2, 0)

    consume(c0 + 1, 1)
    return carry

  lax.fori_loop(0, nchunks // 2, pair, 0)


def _bag_body(pu, pi, uidx, iidx, bagu, bagi, tblu_s, tbli_s,
              idxall, rows_a, rows_bb, out_a, out_bb, sem_a, sem_bb):
  wid = _wid()
  bufs = (idxall, (rows_a, rows_bb), (out_a, out_bb), (sem_a, sem_bb))

  @pl.when(lax.axis_index("s") == 0)
  def _():
    pltpu.sync_copy(pu, tblu_s)
    pltpu.sync_copy(pi, tbli_s)

  plsc.subcore_barrier()

  def run(tbl, fidx, outp):
    _gather_sum_pipeline(
        tbl, fidx, outp, bufs, nchunks=NCHUNK, nodes=CHUNK, r=F,
        idx0=wid * N_PER_W * F, orow0=wid * N_PER_W, bf16_acc=True)

  run(tblu_s, uidx, bagu)
  run(tbli_s, iidx, bagi)


_bag_call = pl.kernel(
    _bag_body,
    out_type=(jax.ShapeDtypeStruct((NU_PAD, D), jnp.bfloat16),
              jax.ShapeDtypeStruct((NU_PAD, D), jnp.bfloat16)),
    mesh=_mesh,
    compiler_params=_sc_params,
    scratch_types=[
        pltpu.VMEM_SHARED((UFEAT_PAD, D), jnp.bfloat16),
        pltpu.VMEM_SHARED((IFEAT_PAD, D), jnp.bfloat16),
        pltpu.VMEM((N_PER_W * F,), jnp.int32),
        pltpu.VMEM((CHUNK * F, D), jnp.bfloat16),
        pltpu.VMEM((CHUNK * F, D), jnp.bfloat16),
        pltpu.VMEM((CHUNK, D), jnp.bfloat16),
        pltpu.VMEM((CHUNK, D), jnp.bfloat16),
        pltpu.SemaphoreType.DMA,
        pltpu.SemaphoreType.DMA,
    ],
)


def _agg_body(bagu, bagi, n0r, n1r, n2r, h0s, g1s, t2s,
              idxall, rows_a, rows_bb, out_a, out_bb, sem_a, sem_bb):
  wid = _wid()
  bufs = (idxall, (rows_a, rows_bb), (out_a, out_bb), (sem_a, sem_bb))

  _gather_sum_pipeline(
      bagu, n2r, t2s, bufs, nchunks=16, nodes=8, r=100,
      idx0=wid * 128 * 100, orow0=wid * 128)

  _gather_sum_pipeline(
      bagi, n1r, g1s, bufs, nchunks=8, nodes=16, r=K,
      idx0=wid * 128 * K, orow0=wid * 128)

  pltpu.sync_copy(n0r.at[pl.ds(wid * 128, 128)], idxall.at[pl.ds(0, 128)])
  for g in range(2):
    pltpu.async_copy(bagu.at[idxall.at[pl.ds(g * 64, 64)]],
                     rows_a.at[pl.ds(g * 64, 64)], sem_a)
  for g in range(2):
    pltpu.make_async_copy(bagu.at[idxall.at[pl.ds(g * 64, 64)]],
                          rows_a.at[pl.ds(g * 64, 64)], sem_a).wait()
  pltpu.sync_copy(rows_a.at[pl.ds(0, 128)], h0s.at[pl.ds(wid * 128, 128)])


_agg_call = pl.kernel(
    _agg_body,
    out_type=(jax.ShapeDtypeStruct((B, D), jnp.bfloat16),
              jax.ShapeDtypeStruct((B, D), jnp.bfloat16),
              jax.ShapeDtypeStruct((B, D), jnp.bfloat16)),
    mesh=_mesh,
    compiler_params=_sc_params,
    scratch_types=[
        pltpu.VMEM((12800,), jnp.int32),
        pltpu.VMEM((800, D), jnp.bfloat16),
        pltpu.VMEM((800, D), jnp.bfloat16),
        pltpu.VMEM((16, D), jnp.bfloat16),
        pltpu.VMEM((16, D), jnp.bfloat16),
        pltpu.SemaphoreType.DMA,
        pltpu.SemaphoreType.DMA,
    ],
)


def _dg(a, b):
  return lax.dot_general(a, b, (((1,), (1,)), ((), ())),
                         preferred_element_type=jnp.float32)


def _proj_body(eu_ref, wu_ref, ei_ref, wi_ref, ou_ref, oi_ref):
  ou_ref[...] = (_dg(eu_ref[...], wu_ref[...]) * (1.0 / F)).astype(jnp.bfloat16)
  oi_ref[...] = (_dg(ei_ref[...], wi_ref[...]) * (1.0 / F)).astype(jnp.bfloat16)


def _proj2(eu, wu, ei, wi):
  eu = jnp.pad(eu, ((0, UFEAT_PAD - UFEAT), (0, 0)))
  ei = jnp.pad(ei, ((0, IFEAT_PAD - IFEAT), (0, 0)))
  return pl.pallas_call(
      _proj_body,
      out_shape=(jax.ShapeDtypeStruct((UFEAT_PAD, D), jnp.bfloat16),
                 jax.ShapeDtypeStruct((IFEAT_PAD, D), jnp.bfloat16)),
  )(eu, wu, ei, wi)


def _final_body(h0_ref, g1_ref, t2_ref, w0_ref, w1_ref,
                bu_ref, bi_ref, b0_ref, b1_ref, o_ref):
  h0 = h0_ref[...].astype(jnp.float32) + bu_ref[...]
  g1 = g1_ref[...].astype(jnp.float32) + float(K) * bi_ref[...]
  t2 = t2_ref[...].astype(jnp.float32) + float(K * K) * bu_ref[...]
  w0 = w0_ref[...]
  w1 = w1_ref[...]
  w0a, w0b = w0[:, :D], w0[:, D:]
  w1a, w1b = w1[:, :D], w1[:, D:]
  y0 = _dg(h0, w0a) + _dg(g1, w0b) + b0_ref[...]
  z = _dg(g1, w0a) + _dg(t2, w0b) + float(K) * b0_ref[...]
  o_ref[...] = _dg(y0, w1a) + _dg(z, w1b) + b1_ref[...]


_final = pl.pallas_call(
    _final_body,
    out_shape=jax.ShapeDtypeStruct((B, D), jnp.float32),
)


@jax.jit
def kernel(n0, n1, n2, user_feat_idx, item_feat_idx, user_feat_emb,
           item_feat_emb, user_proj_w, user_proj_b, item_proj_w, item_proj_b,
           w0_w, w0_b, w1_w, w1_b):
  pu, pi = _proj2(user_feat_emb, user_proj_w, item_feat_emb, item_proj_w)
  uidx = jnp.pad(user_feat_idx, (0, (NU_PAD - NU) * F))
  iidx = jnp.pad(item_feat_idx, (0, (NU_PAD - NI) * F))
  bagu, bagi = _bag_call(pu, pi, uidx, iidx)
  h0s, g1s, t2s = _agg_call(bagu, bagi, n0, n1, n2)
  return _final(h0s, g1s, t2s, w0_w, w1_w,
                user_proj_b.reshape(1, D), item_proj_b.reshape(1, D),
                w0_b.reshape(1, D), w1_b.reshape(1, D))
